# baseline (speedup 1.0000x reference)
import functools

import jax
import jax.numpy as jnp
import numpy as np
from jax import lax
from jax.experimental import pallas as pl
from jax.experimental.pallas import tpu as pltpu
from jax.experimental.pallas import tpu_sc as plsc

_N_BINS = 9
_DELTA = 180.0 / _N_BINS
_EPS = 1e-09
_RAD2DEG = 180.0 / np.pi

_NC, _NS, _L = 2, 16, 16


def _splat_i32(v):
    return jnp.full((_L,), v, jnp.int32)


def _bf16_round(v):
    u = plsc.bitcast(v, jnp.uint32)
    odd = (u >> 16) & jnp.uint32(1)
    u = (u + jnp.uint32(0x7FFF) + odd) & jnp.uint32(0xFFFF0000)
    return plsc.bitcast(u, jnp.float32)


def _sqrt(s):
    i = plsc.bitcast(s, jnp.int32)
    y = plsc.bitcast(jnp.int32(0x5F3759DF) - (i >> 1), jnp.float32)
    for _ in range(3):
        y = y * (1.5 - 0.5 * s * y * y)
    return s * y


def _abs_atan_deg(t):
    big = t > 2.414213562373095
    mid = t > 0.4142135623730950
    x1 = jnp.where(big, -1.0 / t, jnp.where(mid, (t - 1.0) / (t + 1.0), t))
    base = jnp.where(big, np.pi / 2, jnp.where(mid, np.pi / 4, 0.0))
    z = x1 * x1
    p = ((((8.05374449538e-2 * z - 1.38776856032e-1) * z
           + 1.99777106478e-1) * z - 3.33329491539e-1) * z * x1 + x1)
    return (base + p) * _RAD2DEG


def _sc_body(x_hbm, out_hbm, buf, hist, ebuf, orow, sem):
    t = lax.axis_index("c") * _NS + lax.axis_index("s")

    offs = []
    copies = []
    for r in range(3):
        k = 2 * t + r
        rs = jnp.minimum(8 * k + 6, 509)
        offs.append(8 * k + 6 - rs)
        for c in range(3):
            copies.append(pltpu.async_copy(
                x_hbm.at[pl.ds(c * 512 + rs, 3)], buf.at[c, r], sem))

    zeros = jnp.zeros((_L,), jnp.float32)
    for i in range(3 * 64 * _N_BINS // _L):
        hist[pl.ds(i * _L, _L)] = zeros
    for cp in copies:
        cp.wait()

    lane = lax.iota(jnp.int32, _L)

    for r in range(3):
        k = 2 * t + r

        def _cell_row(r=r, k=k):
            off = offs[r]
            up_row = _splat_i32(off)
            mid_row = _splat_i32(off + 1)
            dn_row = _splat_i32(jnp.minimum(off + 2, 2))
            k_is_last = (k == 63)
            for q in range(4):
                col = (lane + 16 * q) * 8
                c6 = col + 6
                c7 = col + 7
                c8 = jnp.minimum(col + 8, 511)
                up = zeros
                left = zeros
                right = zeros
                down = zeros
                for c in range(3):
                    cc = _splat_i32(c)
                    rr = _splat_i32(r)
                    up = up + _bf16_round(
                        plsc.load_gather(buf, [cc, rr, up_row, c7]))
                    left = left + _bf16_round(
                        plsc.load_gather(buf, [cc, rr, mid_row, c6]))
                    right = right + _bf16_round(
                        plsc.load_gather(buf, [cc, rr, mid_row, c8]))
                    down = down + _bf16_round(
                        plsc.load_gather(buf, [cc, rr, dn_row, c7]))
                down = jnp.where(k_is_last, zeros, down)
                right = jnp.where(col + 8 > 511, zeros, right)

                gv = down - up
                gh = right - left
                mag = _sqrt(gv * gv + gh * gh + 1e-06)
                ang = _abs_atan_deg(jnp.abs(gh / (gv + _EPS)))

                t2 = ang * (1.0 / _DELTA) - 0.5
                jb_i = t2.astype(jnp.int32)
                jbin = jnp.where(t2 < 0.0, -1, jb_i)
                jbf = jbin.astype(jnp.float32)
                c_j = _DELTA * (jbf + 1.5)
                vj = mag * ((c_j - ang) * (1.0 / _DELTA))
                vj1 = mag - vj
                idx0 = jnp.where(jbin < 0, jbin + _N_BINS, jbin)
                idx1 = jbin + 1

                hbase = (_splat_i32(r * 64 + 16 * q) + lane) * _N_BINS
                plsc.store_scatter(hist, [hbase + idx0], vj)
                plsc.store_scatter(hist, [hbase + idx1], vj1)
                ebuf[pl.ds(r * 64 + 16 * q, _L)] = vj * vj + vj1 * vj1

        if r < 2:
            _cell_row()
        else:
            pl.when(t < 31)(_cell_row)

    for i in range(2):
        gi = 2 * t + i

        def _out_row(i=i, gi=gi):
            for q in range(4):
                j = lane + 16 * q
                jp = jnp.minimum(j + 1, 63)
                e00 = plsc.load_gather(ebuf, [_splat_i32(i * 64) + j])
                e01 = plsc.load_gather(ebuf, [_splat_i32(i * 64) + jp])
                e10 = plsc.load_gather(ebuf, [_splat_i32((i + 1) * 64) + j])
                e11 = plsc.load_gather(ebuf, [_splat_i32((i + 1) * 64) + jp])
                en = e00 + e01 + e10 + e11
                inv = 1.0 / (_sqrt(en) + _EPS)
                for qd in range(4):
                    di, dj = qd >> 1, qd & 1
                    jj = j if dj == 0 else jp
                    cbase = (_splat_i32((i + di) * 64) + jj) * _N_BINS
                    for b in range(_N_BINS):
                        v = plsc.load_gather(hist, [cbase + _splat_i32(b)])
                        plsc.store_scatter(
                            orow, [j, _splat_i32(qd * _N_BINS + b)], v * inv)
            pltpu.sync_copy(orow, out_hbm.at[gi])

        if i == 0:
            _out_row()
        else:
            pl.when(gi <= 62)(_out_row)


@jax.jit
def _hog_sc(x2):
    mesh = plsc.VectorSubcoreMesh(core_axis_name="c", subcore_axis_name="s",
                                  num_cores=_NC, num_subcores=_NS)
    f = pl.kernel(
        _sc_body,
        out_type=jax.ShapeDtypeStruct((63, 64, 36), jnp.float32),
        mesh=mesh,
        scratch_types=[
            pltpu.VMEM((3, 3, 3, 512), jnp.float32),
            pltpu.VMEM((3 * 64 * _N_BINS,), jnp.float32),
            pltpu.VMEM((3 * 64,), jnp.float32),
            pltpu.VMEM((64, 36), jnp.float32),
            pltpu.SemaphoreType.DMA,
        ],
        compiler_params=pltpu.CompilerParams(
            use_tc_tiling_on_sc=False, needs_layout_passes=False),
    )
    return f(x2)


def kernel(x, W_v, W_h):
    x2 = x.reshape(3 * 512, 512)
    out = _hog_sc(x2)
    return out[:, :63, :], 63, 63


# speedup vs baseline: 1.1674x; 1.1674x over previous
import jax
import jax.numpy as jnp
import numpy as np
from jax import lax
from jax.experimental import pallas as pl
from jax.experimental.pallas import tpu as pltpu
from jax.experimental.pallas import tpu_sc as plsc

_N_BINS = 9
_DELTA = 180.0 / _N_BINS
_EPS = 1e-09
_RAD2DEG = 180.0 / np.pi

_NC, _NS, _L = 2, 16, 16
_HP = 192


def _splat_i32(v):
    return jnp.full((_L,), v, jnp.int32)


def _bf16_round(v):
    u = plsc.bitcast(v, jnp.uint32)
    odd = (u >> 16) & jnp.uint32(1)
    u = (u + jnp.uint32(0x7FFF) + odd) & jnp.uint32(0xFFFF0000)
    return plsc.bitcast(u, jnp.float32)


def _sqrt(s):
    i = plsc.bitcast(s, jnp.int32)
    y = plsc.bitcast(jnp.int32(0x5F3759DF) - (i >> 1), jnp.float32)
    for _ in range(2):
        y = y * (1.5 - 0.5 * s * y * y)
    return s * y


def _abs_atan_deg(t):
    big = t > 2.414213562373095
    mid = t > 0.4142135623730950
    x1 = jnp.where(big, -1.0 / t, jnp.where(mid, (t - 1.0) / (t + 1.0), t))
    base = jnp.where(big, np.pi / 2, jnp.where(mid, np.pi / 4, 0.0))
    z = x1 * x1
    p = ((((8.05374449538e-2 * z - 1.38776856032e-1) * z
           + 1.99777106478e-1) * z - 3.33329491539e-1) * z * x1 + x1)
    return (base + p) * _RAD2DEG


def _sc_body(x_hbm, out_hbm, buf, hist, ebuf, orow, sem):
    t = lax.axis_index("c") * _NS + lax.axis_index("s")

    offs = []
    copies = []
    for r in range(3):
        k = 2 * t + r
        rs = jnp.minimum(8 * k + 6, 509)
        offs.append(jnp.minimum(8 * k + 6 - rs, 1))
        for c in range(3):
            copies.append(pltpu.async_copy(
                x_hbm.at[pl.ds(c * 512 + rs, 3)], buf.at[c, r], sem))

    zeros = jnp.zeros((_L,), jnp.float32)

    def _zero(i, _):
        hist[pl.ds(i * _L, _L)] = zeros
        return 0

    lax.fori_loop(0, (_N_BINS * _HP + _L) // _L, _zero, 0, unroll=4)
    for cp in copies:
        cp.wait()

    lane = lax.iota(jnp.int32, _L)

    def _grad(it, _):
        r = it >> 2
        q = it & 3
        k = 2 * t + r
        off = jnp.where(r == 0, offs[0],
                        jnp.where(r == 1, offs[1], offs[2]))
        rr = _splat_i32(r)
        up_row = _splat_i32(off)
        mid_row = _splat_i32(off + 1)
        dn_row = _splat_i32(jnp.minimum(off + 2, 2))
        col = (lane + 16 * q) * 8
        c6 = col + 6
        c7 = col + 7
        c8 = jnp.minimum(col + 8, 511)
        up = zeros
        left = zeros
        right = zeros
        down = zeros
        for c in range(3):
            cc = _splat_i32(c)
            up = up + _bf16_round(plsc.load_gather(buf, [cc, rr, up_row, c7]))
            left = left + _bf16_round(plsc.load_gather(buf, [cc, rr, mid_row, c6]))
            right = right + _bf16_round(plsc.load_gather(buf, [cc, rr, mid_row, c8]))
            down = down + _bf16_round(plsc.load_gather(buf, [cc, rr, dn_row, c7]))
        down = jnp.where(k == 63, zeros, down)
        right = jnp.where(col + 8 > 511, zeros, right)

        gv = down - up
        gh = right - left
        mag = _sqrt(gv * gv + gh * gh + 1e-06)
        ang = _abs_atan_deg(jnp.abs(gh / (gv + _EPS)))

        t2 = ang * (1.0 / _DELTA) - 0.5
        jb_i = t2.astype(jnp.int32)
        jbin = jnp.where(t2 < 0.0, -1, jb_i)
        jbf = jbin.astype(jnp.float32)
        c_j = _DELTA * (jbf + 1.5)
        vj = mag * ((c_j - ang) * (1.0 / _DELTA))
        vj1 = mag - vj
        idx0 = jnp.where(jbin < 0, jbin + _N_BINS, jbin)
        idx1 = jbin + 1

        cell = _splat_i32(r * 64 + 16 * q) + lane
        plsc.store_scatter(hist, [idx0 * _HP + cell], vj)
        plsc.store_scatter(hist, [idx1 * _HP + cell], vj1)
        ebuf[pl.ds(r * 64 + 16 * q, _L)] = vj * vj + vj1 * vj1
        return 0

    n_grad = jnp.where(t < 31, 12, 8)
    lax.fori_loop(0, n_grad, _grad, 0)

    def _chunk(i, q):
        base = i * 64 + 16 * q
        e00 = ebuf[pl.ds(base, _L)]
        e01 = ebuf[pl.ds(base + 1, _L)]
        e10 = ebuf[pl.ds(base + 64, _L)]
        e11 = ebuf[pl.ds(base + 65, _L)]
        inv = 1.0 / (_sqrt(e00 + e01 + e10 + e11) + _EPS)
        jvec = lane + 16 * q
        for qd in range(4):
            di, dj = qd >> 1, qd & 1
            hb = (i + di) * 64 + dj + 16 * q
            for b in range(_N_BINS):
                v = hist[pl.ds(b * _HP + hb, _L)]
                plsc.store_scatter(
                    orow, [jvec, _splat_i32(qd * _N_BINS + b)], v * inv)
        return 0

    for i in range(2):
        gi = 2 * t + i

        def _out_row(i=i, gi=gi):
            lax.fori_loop(0, 4, lambda q, _: _chunk(i, q), 0)
            pltpu.sync_copy(orow, out_hbm.at[gi])

        if i == 0:
            _out_row()
        else:
            pl.when(gi <= 62)(_out_row)


@jax.jit
def _hog_sc(x2):
    mesh = plsc.VectorSubcoreMesh(core_axis_name="c", subcore_axis_name="s",
                                  num_cores=_NC, num_subcores=_NS)
    f = pl.kernel(
        _sc_body,
        out_type=jax.ShapeDtypeStruct((63, 64, 36), jnp.float32),
        mesh=mesh,
        scratch_types=[
            pltpu.VMEM((3, 3, 3, 512), jnp.float32),
            pltpu.VMEM((_N_BINS * _HP + _L,), jnp.float32),
            pltpu.VMEM((3 * 64 + _L,), jnp.float32),
            pltpu.VMEM((64, 36), jnp.float32),
            pltpu.SemaphoreType.DMA,
        ],
        compiler_params=pltpu.CompilerParams(
            use_tc_tiling_on_sc=False, needs_layout_passes=False),
    )
    return f(x2)


def kernel(x, W_v, W_h):
    x2 = x.reshape(3 * 512, 512)
    out = _hog_sc(x2)
    return out[:, :63, :], 63, 63
